# baseline (speedup 1.0000x reference)
import functools

import jax
import jax.numpy as jnp
from jax import lax
from jax.experimental import pallas as pl
from jax.experimental.pallas import tpu as pltpu
from jax.experimental.pallas import tpu_sc as plsc

N_NODES = 10000
N_EDGES = 320000
D = 128
D_EDGE = 16

NC = 2
NS = 16
NW = NC * NS
NP = 10240
ROWS_PER_TILE = NP // NS
CH = 64
NCH = 160
_QS = 16
EPW = NCH * CH
E_PAD = NW * EPW

_sc_mesh = plsc.VectorSubcoreMesh(core_axis_name="c", subcore_axis_name="s")


@functools.partial(
    pl.kernel,
    out_type=jax.ShapeDtypeStruct((NC, 2, NP), jnp.float32),
    mesh=_sc_mesh,
    scratch_types=[
        pltpu.VMEM((NCH, CH), jnp.int32),
        pltpu.VMEM((NCH, CH), jnp.int32),
        pltpu.VMEM((CH,), jnp.float32),
        pltpu.VMEM_SHARED((NP,), jnp.float32),
        pltpu.VMEM_SHARED((NP,), jnp.float32),
        pltpu.SemaphoreType.DMA,
        pltpu.SemaphoreType.DMA,
        pltpu.SemaphoreType.DMA,
        pltpu.SemaphoreType.DMA,
    ],
)
def _sc_degrees(src_hbm, dst_hbm, zeros1_hbm, out_hbm,
                idx_s, idx_d, ones, dsrc_sh, ddst_sh,
                dsem0, dsem1, dsem2, dsem3):
    cid = lax.axis_index("c")
    sid = lax.axis_index("s")
    wid = sid * NC + cid
    dsems = (dsem0, dsem1, dsem2, dsem3)
    for j in range(CH // 16):
        ones[pl.ds(j * 16, 16)] = jnp.full((16,), 1.0, jnp.float32)
    base = sid * ROWS_PER_TILE
    pltpu.sync_copy(zeros1_hbm, dsrc_sh.at[pl.ds(base, ROWS_PER_TILE)])
    pltpu.sync_copy(zeros1_hbm, ddst_sh.at[pl.ds(base, ROWS_PER_TILE)])
    pltpu.sync_copy(src_hbm.at[wid], idx_s)
    pltpu.sync_copy(dst_hbm.at[wid], idx_d)
    plsc.subcore_barrier()
    descs = []
    for ch in range(NCH):
        descs.append(pltpu.async_copy(
            ones, dsrc_sh.at[idx_s.at[ch]], dsems[ch % 4], add=True))
        descs.append(pltpu.async_copy(
            ones, ddst_sh.at[idx_d.at[ch]], dsems[ch % 4], add=True))
    for dsc in descs:
        dsc.wait()
    plsc.subcore_barrier()
    pltpu.sync_copy(dsrc_sh.at[pl.ds(base, ROWS_PER_TILE)],
                    out_hbm.at[cid, 0, pl.ds(base, ROWS_PER_TILE)])
    pltpu.sync_copy(ddst_sh.at[pl.ds(base, ROWS_PER_TILE)],
                    out_hbm.at[cid, 1, pl.ds(base, ROWS_PER_TILE)])


@functools.partial(
    pl.kernel,
    out_type=jax.ShapeDtypeStruct((NC, NP, D), jnp.float32),
    mesh=_sc_mesh,
    scratch_types=[
        pltpu.VMEM((_QS, CH), jnp.int32),
        pltpu.VMEM((_QS, CH), jnp.int32),
        pltpu.VMEM((_QS, CH), jnp.int32),
        pltpu.VMEM((_QS, CH), jnp.int32),
        pltpu.VMEM((CH, D), jnp.float32),
        pltpu.VMEM((CH, D), jnp.float32),
        pltpu.VMEM((CH, D), jnp.float32),
        pltpu.VMEM_SHARED((NP, D), jnp.float32),
        pltpu.SemaphoreType.DMA,
        pltpu.SemaphoreType.DMA,
        pltpu.SemaphoreType.DMA,
        pltpu.SemaphoreType.DMA,
        pltpu.SemaphoreType.DMA,
        pltpu.SemaphoreType.DMA,
        pltpu.SemaphoreType.DMA,
        pltpu.SemaphoreType.DMA,
    ],
)
def _sc_aggregate(src_hbm, dst_hbm, feats_hbm, zeros2_hbm, out_hbm,
                  idx_sA, idx_sB, idx_dA, idx_dB, rows0, rows1, rows2, agg_sh,
                  gsem0, gsem1, gsem2, ssem0, ssem1, ssem2,
                  isem0, isem1):
    cid = lax.axis_index("c")
    sid = lax.axis_index("s")
    wid = sid * NC + cid
    base = sid * ROWS_PER_TILE
    pltpu.sync_copy(zeros2_hbm,
                    agg_sh.at[pl.ds(base, ROWS_PER_TILE)])
    plsc.subcore_barrier()
    NB = 3
    rows = (rows0, rows1, rows2)
    gsems = (gsem0, gsem1, gsem2)
    ssems = (ssem0, ssem1, ssem2)
    isems = (isem0, isem1)
    npass = NCH // _QS
    gdesc = [None] * NB
    sdesc = [None] * NB
    idesc = [None, None]
    slab_waited = [False] * npass

    idx_s = (idx_sA, idx_sB)
    idx_d = (idx_dA, idx_dB)

    def start_slab_load(o):
        p = o % 2
        idesc[p] = pltpu.async_copy(
            src_hbm.at[wid, pl.ds(o * _QS, _QS)], idx_s[p], isems[p])
        pltpu.async_copy(
            dst_hbm.at[wid, pl.ds(o * _QS, _QS)], idx_d[p], isems[p])

    def wait_slab(o):
        if not slab_waited[o]:
            idesc[o % 2].wait()
            idesc[o % 2].wait()
            slab_waited[o] = True

    def start_gather(i):
        wait_slab(i // _QS)
        k = i % NB
        if sdesc[k] is not None:
            sdesc[k].wait()
            sdesc[k] = None
        gdesc[k] = pltpu.async_copy(
            feats_hbm.at[idx_s[(i // _QS) % 2].at[i % _QS]], rows[k], gsems[k])

    start_slab_load(0)
    if npass > 1:
        start_slab_load(1)
    for t in range(NB - 1):
        start_gather(t)
    for t in range(NCH):
        b = t % NB
        gdesc[b].wait()
        sdesc[b] = pltpu.async_copy(
            rows[b], agg_sh.at[idx_d[(t // _QS) % 2].at[t % _QS]],
            ssems[b], add=True)
        if t + NB - 1 < NCH:
            start_gather(t + NB - 1)
        if t % _QS == 0 and t > 0 and t // _QS + 1 < npass:
            start_slab_load(t // _QS + 1)
    for k in range(NB):
        if sdesc[k] is not None:
            sdesc[k].wait()
            sdesc[k] = None
    plsc.subcore_barrier()
    pltpu.sync_copy(agg_sh.at[pl.ds(base, ROWS_PER_TILE)],
                    out_hbm.at[cid, pl.ds(base, ROWS_PER_TILE)])


_N_GRID = 10
_N_BLK = N_NODES // _N_GRID


def _tc_scale_body(feats_ref, deg_ref, out_ref):
    deg_out = deg_ref[:, 0:1] + deg_ref[:, 2:3]
    norm = lax.rsqrt(jnp.maximum(deg_out, 1.0))
    out_ref[...] = feats_ref[...] * norm


def _tc_scale(feats, deg_t):
    return pl.pallas_call(
        _tc_scale_body,
        grid=(_N_GRID,),
        in_specs=[
            pl.BlockSpec((_N_BLK, D), lambda i: (i, 0)),
            pl.BlockSpec((_N_BLK, 4), lambda i: (i, 0)),
        ],
        out_specs=pl.BlockSpec((_N_BLK, D), lambda i: (i, 0)),
        out_shape=jax.ShapeDtypeStruct((NP, D), jnp.float32),
    )(feats, deg_t)


_E_GRID = 16
_EF_BLK = N_EDGES // _E_GRID


def _tc_edge_body(ef_ref, w_ref, b_ref, es_ref):
    es_ref[...] = (
        jnp.dot(ef_ref[...], w_ref[...], preferred_element_type=jnp.float32)
        + b_ref[...]
    )


def _tc_edge(edge_feat, W_out, b_out2):
    return pl.pallas_call(
        _tc_edge_body,
        grid=(_E_GRID,),
        in_specs=[
            pl.BlockSpec((_EF_BLK, D_EDGE), lambda i: (i, 0)),
            pl.BlockSpec((D_EDGE, 1), lambda i: (0, 0)),
            pl.BlockSpec((1, 1), lambda i: (0, 0)),
        ],
        out_specs=pl.BlockSpec((_EF_BLK, 1), lambda i: (i, 0)),
        out_shape=jax.ShapeDtypeStruct((N_EDGES, 1), jnp.float32),
    )(edge_feat, W_out, b_out2)


def _tc_final_body(agg_ref, deg_ref, w_ref, b_ref, out_ref):
    a = agg_ref[0] + agg_ref[1]
    deg_in = deg_ref[:, 1:2] + deg_ref[:, 3:4]
    norm = lax.rsqrt(jnp.maximum(deg_in, 1.0))
    h = jnp.dot(a * norm, w_ref[...], preferred_element_type=jnp.float32)
    out_ref[...] = h[:N_NODES, :] + b_ref[...]


def _tc_final(agg_part, deg_t, W, b2):
    return pl.pallas_call(
        _tc_final_body,
        out_shape=jax.ShapeDtypeStruct((N_NODES, D), jnp.float32),
    )(agg_part, deg_t, W, b2)


def kernel(feats, edge_index, edge_feat, W, b, W_out, b_out):
    src = edge_index[0].astype(jnp.int32)
    dst = edge_index[1].astype(jnp.int32)

    n_pad = E_PAD - N_EDGES
    ar = jnp.arange(n_pad, dtype=jnp.int32)
    pad_src = N_NODES + (ar % (NP - N_NODES))
    pad_dst = N_NODES + (ar % (NP - N_NODES))
    src3 = jnp.concatenate([src, pad_src]).reshape(NW, NCH, CH)
    dst3 = jnp.concatenate([dst, pad_dst]).reshape(NW, NCH, CH)

    zeros1 = jnp.zeros((ROWS_PER_TILE,), jnp.float32)
    zeros2 = jnp.zeros((ROWS_PER_TILE, D), jnp.float32)

    deg_part = _sc_degrees(src3, dst3, zeros1)
    deg_t = deg_part.reshape(2 * NC, NP).T

    feats_scaled = _tc_scale(feats, deg_t)
    agg_part = _sc_aggregate(src3, dst3, feats_scaled, zeros2)

    h = _tc_final(agg_part, deg_t, W, b.reshape(1, D))
    edge_scores = _tc_edge(edge_feat, W_out, b_out.reshape(1, 1))
    return (edge_scores, h)


# speedup vs baseline: 2.1570x; 2.1570x over previous
import functools

import jax
import jax.numpy as jnp
from jax import lax
from jax.experimental import pallas as pl
from jax.experimental.pallas import tpu as pltpu
from jax.experimental.pallas import tpu_sc as plsc

N_NODES = 10000
N_EDGES = 320000
D = 128
D_EDGE = 16

NC = 2
NS = 16
NW = NC * NS
NP = 10240
ROWS_PER_TILE = NP // NS
CH = 64
NCH = 160
_QS = 16
EPW = NCH * CH
E_PAD = NW * EPW

_sc_mesh = plsc.VectorSubcoreMesh(core_axis_name="c", subcore_axis_name="s")


@functools.partial(
    pl.kernel,
    out_type=jax.ShapeDtypeStruct((NC, 2, NP), jnp.float32),
    mesh=_sc_mesh,
    scratch_types=[
        pltpu.VMEM((NCH, CH), jnp.int32),
        pltpu.VMEM((NCH, CH), jnp.int32),
        pltpu.VMEM((CH,), jnp.float32),
        pltpu.VMEM((ROWS_PER_TILE,), jnp.float32),
        pltpu.VMEM_SHARED((NP,), jnp.float32),
        pltpu.VMEM_SHARED((NP,), jnp.float32),
        pltpu.SemaphoreType.DMA,
        pltpu.SemaphoreType.DMA,
        pltpu.SemaphoreType.DMA,
        pltpu.SemaphoreType.DMA,
    ],
)
def _sc_degrees(src_hbm, dst_hbm, out_hbm,
                idx_s, idx_d, ones, zbuf, dsrc_sh, ddst_sh,
                dsem0, dsem1, dsem2, dsem3):
    cid = lax.axis_index("c")
    sid = lax.axis_index("s")
    wid = sid * NC + cid
    dsems = (dsem0, dsem1, dsem2, dsem3)
    for j in range(CH // 16):
        ones[pl.ds(j * 16, 16)] = jnp.full((16,), 1.0, jnp.float32)

    def _zb(j, _):
        zbuf[pl.ds(j * 16, 16)] = jnp.zeros((16,), jnp.float32)
        return 0

    lax.fori_loop(0, ROWS_PER_TILE // 16, _zb, 0)
    base = sid * ROWS_PER_TILE
    pltpu.sync_copy(zbuf, dsrc_sh.at[pl.ds(base, ROWS_PER_TILE)])
    pltpu.sync_copy(zbuf, ddst_sh.at[pl.ds(base, ROWS_PER_TILE)])
    pltpu.sync_copy(src_hbm.at[wid], idx_s)
    pltpu.sync_copy(dst_hbm.at[wid], idx_d)
    plsc.subcore_barrier()
    descs = []
    for ch in range(NCH):
        descs.append(pltpu.async_copy(
            ones, dsrc_sh.at[idx_s.at[ch]], dsems[ch % 4], add=True))
        descs.append(pltpu.async_copy(
            ones, ddst_sh.at[idx_d.at[ch]], dsems[ch % 4], add=True))
    for dsc in descs:
        dsc.wait()
    plsc.subcore_barrier()
    pltpu.sync_copy(dsrc_sh.at[pl.ds(base, ROWS_PER_TILE)],
                    out_hbm.at[cid, 0, pl.ds(base, ROWS_PER_TILE)])
    pltpu.sync_copy(ddst_sh.at[pl.ds(base, ROWS_PER_TILE)],
                    out_hbm.at[cid, 1, pl.ds(base, ROWS_PER_TILE)])


@functools.partial(
    pl.kernel,
    out_type=jax.ShapeDtypeStruct((NC, NP, D), jnp.float32),
    mesh=_sc_mesh,
    scratch_types=[
        pltpu.VMEM((_QS, CH), jnp.int32),
        pltpu.VMEM((_QS, CH), jnp.int32),
        pltpu.VMEM((_QS, CH), jnp.int32),
        pltpu.VMEM((_QS, CH), jnp.int32),
        pltpu.VMEM((CH, D), jnp.float32),
        pltpu.VMEM((CH, D), jnp.float32),
        pltpu.VMEM((CH, D), jnp.float32),
        pltpu.VMEM_SHARED((NP, D), jnp.float32),
        pltpu.SemaphoreType.DMA,
        pltpu.SemaphoreType.DMA,
        pltpu.SemaphoreType.DMA,
        pltpu.SemaphoreType.DMA,
        pltpu.SemaphoreType.DMA,
        pltpu.SemaphoreType.DMA,
        pltpu.SemaphoreType.DMA,
        pltpu.SemaphoreType.DMA,
    ],
)
def _sc_aggregate(src_hbm, dst_hbm, feats_hbm, out_hbm,
                  idx_sA, idx_sB, idx_dA, idx_dB, rows0, rows1, rows2, agg_sh,
                  gsem0, gsem1, gsem2, ssem0, ssem1, ssem2,
                  isem0, isem1):
    cid = lax.axis_index("c")
    sid = lax.axis_index("s")
    wid = sid * NC + cid
    base = sid * ROWS_PER_TILE

    def _zr(r, _):
        for j in range(D // 16):
            rows0[r, pl.ds(j * 16, 16)] = jnp.zeros((16,), jnp.float32)
        return 0

    lax.fori_loop(0, CH, _zr, 0)
    for r in range(ROWS_PER_TILE // CH):
        pltpu.sync_copy(rows0, agg_sh.at[pl.ds(base + r * CH, CH)])
    plsc.subcore_barrier()
    NB = 3
    rows = (rows0, rows1, rows2)
    gsems = (gsem0, gsem1, gsem2)
    ssems = (ssem0, ssem1, ssem2)
    isems = (isem0, isem1)
    npass = NCH // _QS
    gdesc = [None] * NB
    sdesc = [None] * NB
    idesc = [None, None]
    slab_waited = [False] * npass

    idx_s = (idx_sA, idx_sB)
    idx_d = (idx_dA, idx_dB)

    def start_slab_load(o):
        p = o % 2
        idesc[p] = pltpu.async_copy(
            src_hbm.at[wid, pl.ds(o * _QS, _QS)], idx_s[p], isems[p])
        pltpu.async_copy(
            dst_hbm.at[wid, pl.ds(o * _QS, _QS)], idx_d[p], isems[p])

    def wait_slab(o):
        if not slab_waited[o]:
            idesc[o % 2].wait()
            idesc[o % 2].wait()
            slab_waited[o] = True

    def start_gather(i):
        wait_slab(i // _QS)
        k = i % NB
        if sdesc[k] is not None:
            sdesc[k].wait()
            sdesc[k] = None
        gdesc[k] = pltpu.async_copy(
            feats_hbm.at[idx_s[(i // _QS) % 2].at[i % _QS]], rows[k], gsems[k])

    start_slab_load(0)
    if npass > 1:
        start_slab_load(1)
    for t in range(NB - 1):
        start_gather(t)
    for t in range(NCH):
        b = t % NB
        gdesc[b].wait()
        sdesc[b] = pltpu.async_copy(
            rows[b], agg_sh.at[idx_d[(t // _QS) % 2].at[t % _QS]],
            ssems[b], add=True)
        if t + NB - 1 < NCH:
            start_gather(t + NB - 1)
        if t % _QS == 0 and t > 0 and t // _QS + 1 < npass:
            start_slab_load(t // _QS + 1)
    for k in range(NB):
        if sdesc[k] is not None:
            sdesc[k].wait()
            sdesc[k] = None
    plsc.subcore_barrier()
    pltpu.sync_copy(agg_sh.at[pl.ds(base, ROWS_PER_TILE)],
                    out_hbm.at[cid, pl.ds(base, ROWS_PER_TILE)])


_N_GRID = 10
_N_BLK = N_NODES // _N_GRID


def _tc_scale_body(feats_ref, deg_ref, out_ref):
    deg_out = deg_ref[:, 0:1] + deg_ref[:, 2:3]
    norm = lax.rsqrt(jnp.maximum(deg_out, 1.0))
    out_ref[...] = feats_ref[...] * norm


def _tc_scale(feats, deg_t):
    return pl.pallas_call(
        _tc_scale_body,
        grid=(_N_GRID,),
        in_specs=[
            pl.BlockSpec((_N_BLK, D), lambda i: (i, 0)),
            pl.BlockSpec((_N_BLK, 4), lambda i: (i, 0)),
        ],
        out_specs=pl.BlockSpec((_N_BLK, D), lambda i: (i, 0)),
        out_shape=jax.ShapeDtypeStruct((NP, D), jnp.float32),
    )(feats, deg_t)


_E_GRID = 10
_EF_BLK = N_EDGES // _E_GRID


def _tc_edge_body(eft_ref, w_ref, b_ref, es_ref):
    es_ref[...] = (
        jnp.sum(eft_ref[...] * w_ref[...], axis=0, keepdims=True)
        + b_ref[...]
    )


def _tc_edge(ef_t, W_out, b_out2):
    return pl.pallas_call(
        _tc_edge_body,
        grid=(_E_GRID,),
        in_specs=[
            pl.BlockSpec((D_EDGE, _EF_BLK), lambda i: (0, i)),
            pl.BlockSpec((D_EDGE, 1), lambda i: (0, 0)),
            pl.BlockSpec((1, 1), lambda i: (0, 0)),
        ],
        out_specs=pl.BlockSpec((1, _EF_BLK), lambda i: (0, i)),
        out_shape=jax.ShapeDtypeStruct((1, N_EDGES), jnp.float32),
    )(ef_t, W_out, b_out2)


def _tc_final_body(agg_ref, deg_ref, w_ref, b_ref, out_ref):
    a = agg_ref[0] + agg_ref[1]
    deg_in = deg_ref[:, 1:2] + deg_ref[:, 3:4]
    norm = lax.rsqrt(jnp.maximum(deg_in, 1.0))
    h = jnp.dot(a * norm, w_ref[...], preferred_element_type=jnp.float32)
    out_ref[...] = h[:N_NODES, :] + b_ref[...]


def _tc_final(agg_part, deg_t, W, b2):
    return pl.pallas_call(
        _tc_final_body,
        out_shape=jax.ShapeDtypeStruct((N_NODES, D), jnp.float32),
    )(agg_part, deg_t, W, b2)


def kernel(feats, edge_index, edge_feat, W, b, W_out, b_out):
    src = edge_index[0].astype(jnp.int32)
    dst = edge_index[1].astype(jnp.int32)

    n_pad = E_PAD - N_EDGES
    ar = jnp.arange(n_pad, dtype=jnp.int32)
    pad_src = N_NODES + (ar % (NP - N_NODES))
    pad_dst = N_NODES + (ar % (NP - N_NODES))
    src3 = jnp.concatenate([src, pad_src]).reshape(NW, NCH, CH)
    dst3 = jnp.concatenate([dst, pad_dst]).reshape(NW, NCH, CH)

    deg_part = _sc_degrees(src3, dst3)
    deg_t = deg_part.reshape(2 * NC, NP).T

    feats_scaled = _tc_scale(feats, deg_t)
    agg_part = _sc_aggregate(src3, dst3, feats_scaled)

    h = _tc_final(agg_part, deg_t, W, b.reshape(1, D))
    es_row = _tc_edge(edge_feat.T, W_out, b_out.reshape(1, 1))
    edge_scores = es_row.reshape(N_EDGES, 1)
    return (edge_scores, h)


# speedup vs baseline: 2.1975x; 1.0188x over previous
import functools

import jax
import jax.numpy as jnp
from jax import lax
from jax.experimental import pallas as pl
from jax.experimental.pallas import tpu as pltpu
from jax.experimental.pallas import tpu_sc as plsc

N_NODES = 10000
N_EDGES = 320000
D = 128
D_EDGE = 16

NC = 2
NS = 16
NW = NC * NS
NP = 10240
ROWS_PER_TILE = NP // NS
CH = 64
NCH = 160
_QS = 16
CH2 = 128
NCH2 = (NCH * CH) // CH2
EPW = NCH * CH
E_PAD = NW * EPW

_sc_mesh = plsc.VectorSubcoreMesh(core_axis_name="c", subcore_axis_name="s")


@functools.partial(
    pl.kernel,
    out_type=jax.ShapeDtypeStruct((NC, 2, NP), jnp.float32),
    mesh=_sc_mesh,
    scratch_types=[
        pltpu.VMEM((NCH2, CH2), jnp.int32),
        pltpu.VMEM((NCH2, CH2), jnp.int32),
        pltpu.VMEM((CH2,), jnp.float32),
        pltpu.VMEM((ROWS_PER_TILE,), jnp.float32),
        pltpu.VMEM_SHARED((NP,), jnp.float32),
        pltpu.VMEM_SHARED((NP,), jnp.float32),
        pltpu.SemaphoreType.DMA,
        pltpu.SemaphoreType.DMA,
        pltpu.SemaphoreType.DMA,
        pltpu.SemaphoreType.DMA,
    ],
)
def _sc_degrees(src_hbm, dst_hbm, out_hbm,
                idx_s, idx_d, ones, zbuf, dsrc_sh, ddst_sh,
                dsem0, dsem1, dsem2, dsem3):
    cid = lax.axis_index("c")
    sid = lax.axis_index("s")
    wid = sid * NC + cid
    dsems = (dsem0, dsem1, dsem2, dsem3)
    for j in range(CH2 // 16):
        ones[pl.ds(j * 16, 16)] = jnp.full((16,), 1.0, jnp.float32)

    def _zb(j, _):
        zbuf[pl.ds(j * 16, 16)] = jnp.zeros((16,), jnp.float32)
        return 0

    lax.fori_loop(0, ROWS_PER_TILE // 16, _zb, 0)
    pltpu.sync_copy(src_hbm.at[wid], idx_s)
    pltpu.sync_copy(dst_hbm.at[wid], idx_d)
    base = sid * ROWS_PER_TILE
    pltpu.sync_copy(zbuf, dsrc_sh.at[pl.ds(base, ROWS_PER_TILE)])
    pltpu.sync_copy(zbuf, ddst_sh.at[pl.ds(base, ROWS_PER_TILE)])
    plsc.subcore_barrier()
    descs = []
    for ch in range(NCH2):
        descs.append(pltpu.async_copy(
            ones, dsrc_sh.at[idx_s.at[ch]], dsems[ch % 4], add=True))
        descs.append(pltpu.async_copy(
            ones, ddst_sh.at[idx_d.at[ch]], dsems[ch % 4], add=True))
    for dsc in descs:
        dsc.wait()
    plsc.subcore_barrier()
    pltpu.sync_copy(dsrc_sh.at[pl.ds(base, ROWS_PER_TILE)],
                    out_hbm.at[cid, 0, pl.ds(base, ROWS_PER_TILE)])
    pltpu.sync_copy(ddst_sh.at[pl.ds(base, ROWS_PER_TILE)],
                    out_hbm.at[cid, 1, pl.ds(base, ROWS_PER_TILE)])


@functools.partial(
    pl.kernel,
    out_type=jax.ShapeDtypeStruct((NC, NP, D), jnp.float32),
    mesh=_sc_mesh,
    scratch_types=[
        pltpu.VMEM((_QS, CH), jnp.int32),
        pltpu.VMEM((_QS, CH), jnp.int32),
        pltpu.VMEM((_QS, CH), jnp.int32),
        pltpu.VMEM((_QS, CH), jnp.int32),
        pltpu.VMEM((CH, D), jnp.float32),
        pltpu.VMEM((CH, D), jnp.float32),
        pltpu.VMEM((CH, D), jnp.float32),
        pltpu.VMEM_SHARED((NP, D), jnp.float32),
        pltpu.SemaphoreType.DMA,
        pltpu.SemaphoreType.DMA,
        pltpu.SemaphoreType.DMA,
        pltpu.SemaphoreType.DMA,
        pltpu.SemaphoreType.DMA,
        pltpu.SemaphoreType.DMA,
        pltpu.SemaphoreType.DMA,
        pltpu.SemaphoreType.DMA,
    ],
)
def _sc_aggregate(src_hbm, dst_hbm, feats_hbm, out_hbm,
                  idx_sA, idx_sB, idx_dA, idx_dB, rows0, rows1, rows2, agg_sh,
                  gsem0, gsem1, gsem2, ssem0, ssem1, ssem2,
                  isem0, isem1):
    cid = lax.axis_index("c")
    sid = lax.axis_index("s")
    wid = sid * NC + cid
    base = sid * ROWS_PER_TILE

    NB = 3
    rows = (rows0, rows1, rows2)
    gsems = (gsem0, gsem1, gsem2)
    ssems = (ssem0, ssem1, ssem2)
    isems = (isem0, isem1)
    npass = NCH // _QS
    gdesc = [None] * NB
    sdesc = [None] * NB
    idesc = [None, None]
    slab_waited = [False] * npass

    idx_s = (idx_sA, idx_sB)
    idx_d = (idx_dA, idx_dB)

    def start_slab_load(o):
        p = o % 2
        idesc[p] = pltpu.async_copy(
            src_hbm.at[wid, pl.ds(o * _QS, _QS)], idx_s[p], isems[p])
        pltpu.async_copy(
            dst_hbm.at[wid, pl.ds(o * _QS, _QS)], idx_d[p], isems[p])

    def wait_slab(o):
        if not slab_waited[o]:
            idesc[o % 2].wait()
            idesc[o % 2].wait()
            slab_waited[o] = True

    def start_gather(i):
        wait_slab(i // _QS)
        k = i % NB
        if sdesc[k] is not None:
            sdesc[k].wait()
            sdesc[k] = None
        gdesc[k] = pltpu.async_copy(
            feats_hbm.at[idx_s[(i // _QS) % 2].at[i % _QS]], rows[k], gsems[k])

    start_slab_load(0)
    if npass > 1:
        start_slab_load(1)

    def _zr(r, _):
        for j in range(D // 16):
            rows2[r, pl.ds(j * 16, 16)] = jnp.zeros((16,), jnp.float32)
        return 0

    lax.fori_loop(0, CH, _zr, 0)
    for r in range(ROWS_PER_TILE // CH):
        pltpu.sync_copy(rows2, agg_sh.at[pl.ds(base + r * CH, CH)])
    for t in range(NB - 1):
        start_gather(t)
    plsc.subcore_barrier()
    for t in range(NCH):
        b = t % NB
        gdesc[b].wait()
        sdesc[b] = pltpu.async_copy(
            rows[b], agg_sh.at[idx_d[(t // _QS) % 2].at[t % _QS]],
            ssems[b], add=True)
        if t + NB - 1 < NCH:
            start_gather(t + NB - 1)
        if t % _QS == 0 and t > 0 and t // _QS + 1 < npass:
            start_slab_load(t // _QS + 1)
    for k in range(NB):
        if sdesc[k] is not None:
            sdesc[k].wait()
            sdesc[k] = None
    plsc.subcore_barrier()
    pltpu.sync_copy(agg_sh.at[pl.ds(base, ROWS_PER_TILE)],
                    out_hbm.at[cid, pl.ds(base, ROWS_PER_TILE)])


_N_GRID = 10
_N_BLK = N_NODES // _N_GRID


def _tc_scale_body(feats_ref, deg_ref, out_ref):
    deg_out = deg_ref[:, 0:1] + deg_ref[:, 2:3]
    norm = lax.rsqrt(jnp.maximum(deg_out, 1.0))
    out_ref[...] = feats_ref[...] * norm


def _tc_scale(feats, deg_t):
    return pl.pallas_call(
        _tc_scale_body,
        grid=(_N_GRID,),
        in_specs=[
            pl.BlockSpec((_N_BLK, D), lambda i: (i, 0)),
            pl.BlockSpec((_N_BLK, 4), lambda i: (i, 0)),
        ],
        out_specs=pl.BlockSpec((_N_BLK, D), lambda i: (i, 0)),
        out_shape=jax.ShapeDtypeStruct((NP, D), jnp.float32),
    )(feats, deg_t)


_E_GRID = 10
_EF_BLK = N_EDGES // _E_GRID


def _tc_edge_body(eft_ref, w_ref, b_ref, es_ref):
    es_ref[...] = (
        jnp.sum(eft_ref[...] * w_ref[...], axis=0, keepdims=True)
        + b_ref[...]
    )


def _tc_edge(ef_t, W_out, b_out2):
    return pl.pallas_call(
        _tc_edge_body,
        grid=(_E_GRID,),
        in_specs=[
            pl.BlockSpec((D_EDGE, _EF_BLK), lambda i: (0, i)),
            pl.BlockSpec((D_EDGE, 1), lambda i: (0, 0)),
            pl.BlockSpec((1, 1), lambda i: (0, 0)),
        ],
        out_specs=pl.BlockSpec((1, _EF_BLK), lambda i: (0, i)),
        out_shape=jax.ShapeDtypeStruct((1, N_EDGES), jnp.float32),
    )(ef_t, W_out, b_out2)


def _tc_final_body(agg_ref, deg_ref, w_ref, b_ref, out_ref):
    a = agg_ref[0] + agg_ref[1]
    deg_in = deg_ref[:, 1:2] + deg_ref[:, 3:4]
    norm = lax.rsqrt(jnp.maximum(deg_in, 1.0))
    h = jnp.dot(a * norm, w_ref[...], preferred_element_type=jnp.float32)
    out_ref[...] = h[:N_NODES, :] + b_ref[...]


def _tc_final(agg_part, deg_t, W, b2):
    return pl.pallas_call(
        _tc_final_body,
        out_shape=jax.ShapeDtypeStruct((N_NODES, D), jnp.float32),
    )(agg_part, deg_t, W, b2)


def kernel(feats, edge_index, edge_feat, W, b, W_out, b_out):
    src = edge_index[0].astype(jnp.int32)
    dst = edge_index[1].astype(jnp.int32)

    n_pad = E_PAD - N_EDGES
    ar = jnp.arange(n_pad, dtype=jnp.int32)
    pad_src = N_NODES + (ar % (NP - N_NODES))
    pad_dst = N_NODES + (ar % (NP - N_NODES))
    src_p = jnp.concatenate([src, pad_src])
    dst_p = jnp.concatenate([dst, pad_dst])
    src3 = src_p.reshape(NW, NCH, CH)
    dst3 = dst_p.reshape(NW, NCH, CH)
    src2 = src_p.reshape(NW, NCH2, CH2)
    dst2 = dst_p.reshape(NW, NCH2, CH2)

    deg_part = _sc_degrees(src2, dst2)
    deg_t = deg_part.reshape(2 * NC, NP).T

    feats_scaled = _tc_scale(feats, deg_t)
    agg_part = _sc_aggregate(src3, dst3, feats_scaled)

    h = _tc_final(agg_part, deg_t, W, b.reshape(1, D))
    es_row = _tc_edge(edge_feat.T, W_out, b_out.reshape(1, 1))
    edge_scores = es_row.reshape(N_EDGES, 1)
    return (edge_scores, h)
